# baseline (speedup 1.0000x reference)
import jax
import jax.numpy as jnp
from jax import lax
from jax.experimental import pallas as pl
from jax.experimental.pallas import tpu as pltpu
from jax.experimental.pallas import tpu_sc as plsc

_TABLE_ROWS = 256 * 256 * 256
_S = 512 * 512
_P = 4 * _S
_NW = 32
_PW = _P // _NW
_L = 16

_CI = 8192
_NI = _PW // _CI
_CF = 8192
_NF = _PW // _CF


def _wid():
    return lax.axis_index("s") * 2 + lax.axis_index("c")


def _idx_body(img_hbm, idx_hbm, xr, xg, xb, iv):
    wid = _wid()
    batch = wid // 8
    base = (wid % 8) * _PW
    x3 = (xr, xg, xb)

    for k in range(_NI):
        off = base + k * _CI
        for ch in range(3):
            pltpu.sync_copy(img_hbm.at[3 * batch + ch, pl.ds(off, _CI)],
                            x3[ch])

        def body(i, _):
            sl = pl.ds(i * _L, _L)
            rr = (xr[sl] + 1.0) * 127.5
            gg = (xg[sl] + 1.0) * 127.5
            bb = (xb[sl] + 1.0) * 127.5
            t = rr * 65536.0 + gg * 256.0 + bb
            iv[sl] = jnp.clip(t.astype(jnp.int32), 0, _TABLE_ROWS - 1)
            return 0

        lax.fori_loop(0, _CI // _L, body, 0, unroll=4)
        pltpu.sync_copy(iv, idx_hbm.at[pl.ds(batch * _S + off, _CI)])


def _chan_body(ch, img_hbm, idx_hbm, wbh, out_hbm,
               x0, x1, y0, y1, iv0, iv1, wb0, wb1,
               sem_in, sem_g, sem_out):
    wid = _wid()
    batch = wid // 8
    base = (wid % 8) * _PW
    xb_ = (x0, x1)
    yb_ = (y0, y1)
    ivb = (iv0, iv1)
    wbb = (wb0, wb1)

    def in_copies(k, p):
        off = base + k * _CF
        return [
            pltpu.make_async_copy(
                img_hbm.at[3 * batch + ch, pl.ds(off, _CF)], xb_[p], sem_in),
            pltpu.make_async_copy(
                idx_hbm.at[pl.ds(batch * _S + off, _CF)], ivb[p], sem_in),
        ]

    def gathers(p):
        return [pltpu.make_async_copy(wbh.at[ivb[p]], wbb[p], sem_g)]

    def out_copies(k, p):
        off = base + k * _CF
        return [pltpu.make_async_copy(
            yb_[p], out_hbm.at[batch, pl.ds(off, _CF)], sem_out)]

    def compute(p):
        x, y, wb = xb_[p], yb_[p], wbb[p]

        def body(i, _):
            sl = pl.ds(i * _L, _L)
            bits = wb[sl]
            w = plsc.bitcast(bits & jnp.int32(-65536), jnp.float32)
            b = plsc.bitcast(bits << 16, jnp.float32)
            t = w * x[sl] + b
            e = jnp.exp(t + t)
            y[sl] = 1.0 - 2.0 / (e + 1.0)
            return 0

        lax.fori_loop(0, _CF // _L, body, 0, unroll=4)

    for c in in_copies(0, 0):
        c.start()
    for c in in_copies(0, 0):
        c.wait()
    for c in gathers(0):
        c.start()
    if _NF > 1:
        for c in in_copies(1, 1):
            c.start()

    for k in range(_NF):
        p = k % 2
        q = (k + 1) % 2
        if k + 1 < _NF:
            for c in in_copies(k + 1, q):
                c.wait()
            for c in gathers(q):
                c.start()
        for c in gathers(p):
            c.wait()
        if k >= 2:
            for c in out_copies(k - 2, p):
                c.wait()
        compute(p)
        for c in out_copies(k, p):
            c.start()
        if k + 2 < _NF:
            for c in in_copies(k + 2, p):
                c.start()

    for k in range(max(0, _NF - 2), _NF):
        for c in out_copies(k, k % 2):
            c.wait()


@jax.jit
def _run(img2, weight, bias):
    mesh = plsc.VectorSubcoreMesh(core_axis_name="c", subcore_axis_name="s")
    cp = pltpu.CompilerParams(needs_layout_passes=False)

    f_idx = pl.kernel(
        _idx_body,
        out_type=jax.ShapeDtypeStruct((_P,), jnp.int32),
        mesh=mesh, compiler_params=cp,
        scratch_types=(
            [pltpu.VMEM((_CI,), jnp.float32) for _ in range(3)]
            + [pltpu.VMEM((_CI,), jnp.int32)]
        ),
    )

    idxh = f_idx(img2)

    wgt, bis = weight, bias
    slices = []
    for ch in range(3):
        w_c = lax.bitcast_convert_type(
            wgt[:, ch].astype(jnp.bfloat16), jnp.uint16).astype(jnp.uint32)
        b_c = lax.bitcast_convert_type(
            bis[:, ch].astype(jnp.bfloat16), jnp.uint16).astype(jnp.uint32)
        wb_c = lax.bitcast_convert_type((w_c << 16) | b_c, jnp.int32)
        wb_c, wgt, bis = lax.optimization_barrier((wb_c, wgt, bis))
        slices.append(wb_c)

    outs = []
    for ch in range(3):
        f_ch = pl.kernel(
            lambda *a, _ch=ch: _chan_body(_ch, *a),
            out_type=jax.ShapeDtypeStruct((4, _S), jnp.float32),
            mesh=mesh, compiler_params=cp,
            scratch_types=(
                [pltpu.VMEM((_CF,), jnp.float32) for _ in range(4)]
                + [pltpu.VMEM((_CF,), jnp.int32) for _ in range(2)]
                + [pltpu.VMEM((_CF,), jnp.int32) for _ in range(2)]
                + [pltpu.SemaphoreType.DMA for _ in range(3)]
            ),
        )
        outs.append(f_ch(img2, idxh, slices[ch]))
    return jnp.stack(outs, axis=1)


def kernel(img, weight, bias):
    img2 = img.reshape(12, _S)
    out = _run(img2, weight, bias)
    return out.reshape(4, 3, 512, 512)


# speedup vs baseline: 1.0088x; 1.0088x over previous
import jax
import jax.numpy as jnp
from jax import lax
from jax.experimental import pallas as pl
from jax.experimental.pallas import tpu as pltpu
from jax.experimental.pallas import tpu_sc as plsc

_TABLE_ROWS = 256 * 256 * 256
_S = 512 * 512
_P = 4 * _S
_NW = 32
_PW = _P // _NW
_L = 16

_CI = 8192
_NI = _PW // _CI
_CF = 4096
_NF = _PW // _CF


def _wid():
    return lax.axis_index("s") * 2 + lax.axis_index("c")


def _idx_body(img_hbm, idx_hbm, xr, xg, xb, iv):
    wid = _wid()
    batch = wid // 8
    base = (wid % 8) * _PW
    x3 = (xr, xg, xb)

    for k in range(_NI):
        off = base + k * _CI
        for ch in range(3):
            pltpu.sync_copy(img_hbm.at[3 * batch + ch, pl.ds(off, _CI)],
                            x3[ch])

        def body(i, _):
            sl = pl.ds(i * _L, _L)
            rr = (xr[sl] + 1.0) * 127.5
            gg = (xg[sl] + 1.0) * 127.5
            bb = (xb[sl] + 1.0) * 127.5
            t = rr * 65536.0 + gg * 256.0 + bb
            iv[sl] = jnp.clip(t.astype(jnp.int32), 0, _TABLE_ROWS - 1)
            return 0

        lax.fori_loop(0, _CI // _L, body, 0, unroll=4)
        pltpu.sync_copy(iv, idx_hbm.at[pl.ds(batch * _S + off, _CI)])


def _chan_body(ch, img_hbm, idx_hbm, wbh, out_hbm,
               x0, x1, y0, y1, iv0, iv1, wb0, wb1,
               sem_in, sem_g, sem_out):
    wid = _wid()
    batch = wid // 8
    base = (wid % 8) * _PW
    xb_ = (x0, x1)
    yb_ = (y0, y1)
    ivb = (iv0, iv1)
    wbb = (wb0, wb1)

    def in_copies(k, p):
        off = base + k * _CF
        return [
            pltpu.make_async_copy(
                img_hbm.at[3 * batch + ch, pl.ds(off, _CF)], xb_[p], sem_in),
            pltpu.make_async_copy(
                idx_hbm.at[pl.ds(batch * _S + off, _CF)], ivb[p], sem_in),
        ]

    def gathers(p):
        return [pltpu.make_async_copy(wbh.at[ivb[p]], wbb[p], sem_g)]

    def out_copies(k, p):
        off = base + k * _CF
        return [pltpu.make_async_copy(
            yb_[p], out_hbm.at[batch, pl.ds(off, _CF)], sem_out)]

    def compute(p):
        x, y, wb = xb_[p], yb_[p], wbb[p]

        def body(i, _):
            sl = pl.ds(i * _L, _L)
            bits = wb[sl]
            w = plsc.bitcast(bits & jnp.int32(-65536), jnp.float32)
            b = plsc.bitcast(bits << 16, jnp.float32)
            t = w * x[sl] + b
            e = jnp.exp(t + t)
            y[sl] = 1.0 - 2.0 / (e + 1.0)
            return 0

        lax.fori_loop(0, _CF // _L, body, 0, unroll=4)

    for c in in_copies(0, 0):
        c.start()
    for c in in_copies(0, 0):
        c.wait()
    for c in gathers(0):
        c.start()
    if _NF > 1:
        for c in in_copies(1, 1):
            c.start()

    for k in range(_NF):
        p = k % 2
        q = (k + 1) % 2
        if k + 1 < _NF:
            for c in in_copies(k + 1, q):
                c.wait()
            for c in gathers(q):
                c.start()
        for c in gathers(p):
            c.wait()
        if k >= 2:
            for c in out_copies(k - 2, p):
                c.wait()
        compute(p)
        for c in out_copies(k, p):
            c.start()
        if k + 2 < _NF:
            for c in in_copies(k + 2, p):
                c.start()

    for k in range(max(0, _NF - 2), _NF):
        for c in out_copies(k, k % 2):
            c.wait()


@jax.jit
def _run(img2, weight, bias):
    mesh = plsc.VectorSubcoreMesh(core_axis_name="c", subcore_axis_name="s")
    cp = pltpu.CompilerParams(needs_layout_passes=False)

    f_idx = pl.kernel(
        _idx_body,
        out_type=jax.ShapeDtypeStruct((_P,), jnp.int32),
        mesh=mesh, compiler_params=cp,
        scratch_types=(
            [pltpu.VMEM((_CI,), jnp.float32) for _ in range(3)]
            + [pltpu.VMEM((_CI,), jnp.int32)]
        ),
    )

    idxh = f_idx(img2)

    wgt, bis = weight, bias
    slices = []
    for ch in range(3):
        w_c = lax.bitcast_convert_type(
            wgt[:, ch].astype(jnp.bfloat16), jnp.uint16).astype(jnp.uint32)
        b_c = lax.bitcast_convert_type(
            bis[:, ch].astype(jnp.bfloat16), jnp.uint16).astype(jnp.uint32)
        wb_c = lax.bitcast_convert_type((w_c << 16) | b_c, jnp.int32)
        wb_c, wgt, bis = lax.optimization_barrier((wb_c, wgt, bis))
        slices.append(wb_c)

    outs = []
    for ch in range(3):
        f_ch = pl.kernel(
            lambda *a, _ch=ch: _chan_body(_ch, *a),
            out_type=jax.ShapeDtypeStruct((4, _S), jnp.float32),
            mesh=mesh, compiler_params=cp,
            scratch_types=(
                [pltpu.VMEM((_CF,), jnp.float32) for _ in range(4)]
                + [pltpu.VMEM((_CF,), jnp.int32) for _ in range(2)]
                + [pltpu.VMEM((_CF,), jnp.int32) for _ in range(2)]
                + [pltpu.SemaphoreType.DMA for _ in range(3)]
            ),
        )
        outs.append(f_ch(img2, idxh, slices[ch]))
    return jnp.stack(outs, axis=1)


def kernel(img, weight, bias):
    img2 = img.reshape(12, _S)
    out = _run(img2, weight, bias)
    return out.reshape(4, 3, 512, 512)
